# baseline (speedup 1.0000x reference)
import functools

import jax
import jax.numpy as jnp
from jax import lax
from jax.experimental import pallas as pl
from jax.experimental.pallas import tpu as pltpu
from jax.experimental.pallas import tpu_sc as plsc

NU = 50000
DIM = 32
NC = 2
NS = 16
NP = 50688
ROWS_T = NP // NS
RCH = 96
NCH = ROWS_T // RCH
EB = 96
SBI = 16

_mesh = plsc.VectorSubcoreMesh(
    core_axis_name="c", subcore_axis_name="s", num_cores=NC, num_subcores=NS
)
_params = pltpu.CompilerParams(
    use_tc_tiling_on_sc=False, needs_layout_passes=False)


def _make_mega(n_edge_rows):
    et_rows = n_edge_rows // NS
    nsb = et_rows // SBI
    nsb_core = n_edge_rows // SBI

    out_t = jax.ShapeDtypeStruct((NC * NP, DIM), jnp.float32)

    @functools.partial(
        pl.kernel,
        out_type=(out_t, out_t, out_t, out_t),
        mesh=_mesh,
        compiler_params=_params,
        scratch_types=[
            pltpu.VMEM_SHARED((NP, DIM), jnp.float32),
            pltpu.VMEM((2, SBI, EB), jnp.int32),
            pltpu.VMEM((2, EB, DIM), jnp.float32),
            pltpu.VMEM((RCH,), jnp.float32),
            pltpu.SemaphoreType.DMA,
            pltpu.SemaphoreType.DMA,
            pltpu.SemaphoreType.DMA,
            pltpu.SemaphoreType.DMA,
        ],
    )
    def mega(x0, idx_o, idx_e, dinv_o, dinv_e, y1, y2, y3, y4,
             accum, idx_buf, rows_buf, dinv_buf, gsem0, gsem1, ssem0, ssem1):
        c = lax.axis_index("c")
        s = lax.axis_index("s")
        row0 = s * ROWS_T
        gsems = (gsem0, gsem1)
        ssems = (ssem0, ssem1)
        sb0 = c * nsb_core + s * nsb

        tabs = (x0, y1, y2, y3)
        outs = (y1, y2, y3, y4)
        idxs = (idx_o, idx_e, idx_o, idx_e)
        dinvs = (dinv_o, dinv_e, dinv_o, dinv_e)

        for k in range(4):
            tab, out, idx, dinv = tabs[k], outs[k], idxs[k], dinvs[k]

            def zrow(r, _):
                for w in range(DIM // 16):
                    rows_buf[0, r, pl.ds(w * 16, 16)] = jnp.zeros(
                        (16,), jnp.float32)
                return 0
            lax.fori_loop(0, RCH, zrow, 0)

            def zcopy(kk, _):
                pltpu.sync_copy(rows_buf.at[0],
                                accum.at[pl.ds(row0 + kk * RCH, RCH)])
                return 0
            lax.fori_loop(0, NCH, zcopy, 0)
            plsc.subcore_barrier()

            def sb_body(sb, _):
                pltpu.sync_copy(idx.at[sb0 + sb], idx_buf)
                d = pltpu.async_copy(tab.at[idx_buf.at[0, 0]],
                                     rows_buf.at[0], gsem0)
                sdescs = [None] * SBI
                for j in range(SBI):
                    slot = j % 2
                    if j + 1 < SBI:
                        if j >= 1:
                            sdescs[j - 1].wait()
                        d_next = pltpu.async_copy(
                            tab.at[idx_buf.at[0, j + 1]],
                            rows_buf.at[(j + 1) % 2], gsems[(j + 1) % 2])
                    d.wait()
                    sd = pltpu.make_async_copy(
                        rows_buf.at[slot], accum.at[idx_buf.at[1, j]],
                        ssems[slot])
                    sd.start(add=True)
                    sdescs[j] = sd
                    if j + 1 < SBI:
                        d = d_next
                sdescs[SBI - 2].wait()
                sdescs[SBI - 1].wait()
                return 0

            lax.fori_loop(0, nsb, sb_body, 0)
            plsc.subcore_barrier()

            def echunk(kk, _):
                pltpu.sync_copy(accum.at[pl.ds(row0 + kk * RCH, RCH)],
                                rows_buf.at[0])
                pltpu.sync_copy(
                    dinv.at[pl.ds(c * NP + row0 + kk * RCH, RCH)], dinv_buf)

                def row_body(r, _):
                    g = plsc.load_gather(
                        dinv_buf, [jnp.full((16,), r, jnp.int32)])
                    for w in range(DIM // 16):
                        rows_buf[0, r, pl.ds(w * 16, 16)] = (
                            rows_buf[0, r, pl.ds(w * 16, 16)] * g)
                    return 0

                lax.fori_loop(0, RCH, row_body, 0)
                pltpu.sync_copy(
                    rows_buf.at[0],
                    out.at[pl.ds(c * NP + row0 + kk * RCH, RCH)])
                return 0

            lax.fori_loop(0, NCH, echunk, 0)

    return mega


def _make_degree(n_edge_rows):
    et_rows = n_edge_rows // NS
    nsb = et_rows // SBI
    nsb_core = n_edge_rows // SBI

    @functools.partial(
        pl.kernel,
        out_type=jax.ShapeDtypeStruct((NC * NP, 16), jnp.float32),
        mesh=_mesh,
        compiler_params=_params,
        scratch_types=[
            pltpu.VMEM_SHARED((NP, 16), jnp.float32),
            pltpu.VMEM((2, SBI, EB), jnp.int32),
            pltpu.VMEM((EB, 16), jnp.float32),
            pltpu.SemaphoreType.DMA,
        ],
    )
    def degree(idx, out, dcum, idx_buf, ones_buf, dsem):
        c = lax.axis_index("c")
        s = lax.axis_index("s")
        row0 = s * ROWS_T
        sb0 = c * nsb_core + s * nsb

        def zrow(r, _):
            ones_buf[r, pl.ds(0, 16)] = jnp.zeros((16,), jnp.float32)
            return 0
        lax.fori_loop(0, EB, zrow, 0)

        def zcopy(kk, _):
            pltpu.sync_copy(ones_buf,
                            dcum.at[pl.ds(row0 + kk * EB, EB)])
            return 0
        lax.fori_loop(0, ROWS_T // EB, zcopy, 0)
        plsc.subcore_barrier()

        def orow(r, _):
            ones_buf[r, pl.ds(0, 16)] = jnp.ones((16,), jnp.float32)
            return 0
        lax.fori_loop(0, EB, orow, 0)

        def sb_body(sb, _):
            pltpu.sync_copy(idx.at[sb0 + sb], idx_buf)
            descs = []
            for j in range(SBI):
                sd = pltpu.make_async_copy(
                    ones_buf, dcum.at[idx_buf.at[1, j]], dsem)
                sd.start(add=True)
                descs.append(sd)
            for sd in descs:
                sd.wait()
            return 0

        lax.fori_loop(0, nsb, sb_body, 0)
        plsc.subcore_barrier()
        pltpu.sync_copy(dcum.at[pl.ds(row0, ROWS_T)],
                        out.at[pl.ds(c * NP + row0, ROWS_T)])

    return degree


def kernel(user_emb, item_emb, edge_user, edge_item):
    E = edge_user.shape[0]
    egran = NS * SBI * EB
    epad = ((E + egran - 1) // egran) * egran
    n_edge_rows = epad // EB

    eu = edge_user.astype(jnp.int32)
    ei = edge_item.astype(jnp.int32)
    pad = jnp.full((epad - E,), NU, jnp.int32)
    eu = jnp.concatenate([eu, pad])
    ei = jnp.concatenate([ei, pad])

    nsb_all = 2 * n_edge_rows // SBI

    def eidx(src_parts, dst_parts):
        src = jnp.concatenate(src_parts).reshape(nsb_all, 1, SBI, EB)
        dst = jnp.concatenate(dst_parts).reshape(nsb_all, 1, SBI, EB)
        return jnp.concatenate([src, dst], axis=1)

    idx_o = eidx([eu, ei + NP], [ei, eu])
    idx_e = eidx([ei, eu + NP], [eu, ei])

    degree_k = _make_degree(n_edge_rows)
    mega_k = _make_mega(n_edge_rows)

    deg = degree_k(idx_e)
    deg_u = jnp.where(deg[:NU, 0] == 0, 1.0, deg[:NU, 0])
    deg_i = jnp.where(deg[NP:NP + NU, 0] == 0, 1.0, deg[NP:NP + NU, 0])

    zpadn = jnp.zeros((NP - NU,), jnp.float32)
    dinv_u = jnp.concatenate([1.0 / deg_u, zpadn])
    dinv_i = jnp.concatenate([1.0 / deg_i, zpadn])
    dinv_o = jnp.concatenate([dinv_i, dinv_u])
    dinv_e = jnp.concatenate([dinv_u, dinv_i])

    isd_u = deg_u ** -0.5
    isd_i = deg_i ** -0.5
    zpadr = jnp.zeros((NP - NU, DIM), jnp.float32)
    x0 = jnp.concatenate([
        user_emb * isd_u[:, None], zpadr,
        item_emb * isd_i[:, None], zpadr,
    ], axis=0)

    y1, y2, y3, y4 = mega_k(x0, idx_o, idx_e, dinv_o, dinv_e)

    p1 = y1[:NU]
    q1 = y1[NP:NP + NU]
    p2 = y2[:NU]
    q2 = y2[NP:NP + NU]
    p4 = y4[:NU]
    q4 = y4[NP:NP + NU]

    sd_u = jnp.sqrt(deg_u)[:, None]
    sd_i = jnp.sqrt(deg_i)[:, None]
    final_user = (2.0 * user_emb + sd_u * (q1 + 2.0 * p2 + p4)) / 3.0
    final_item = (2.0 * item_emb + sd_i * (p1 + 2.0 * q2 + q4)) / 3.0
    return (final_user, final_item)


# speedup vs baseline: 1.0249x; 1.0249x over previous
import functools

import jax
import jax.numpy as jnp
from jax import lax
from jax.experimental import pallas as pl
from jax.experimental.pallas import tpu as pltpu
from jax.experimental.pallas import tpu_sc as plsc

NU = 50000
DIM = 32
NC = 2
NS = 16
NP = 50688
ROWS_T = NP // NS
RCH = 96
NCH = ROWS_T // RCH
EB = 96
SBI = 16

_mesh = plsc.VectorSubcoreMesh(
    core_axis_name="c", subcore_axis_name="s", num_cores=NC, num_subcores=NS
)
_params = pltpu.CompilerParams(
    use_tc_tiling_on_sc=False, needs_layout_passes=False)


def _make_stage(n_edge_rows):
    et_rows = n_edge_rows // NS
    nsb = et_rows // SBI
    nsb_core = n_edge_rows // SBI

    @functools.partial(
        pl.kernel,
        out_type=jax.ShapeDtypeStruct((NC * NP, DIM), jnp.float32),
        mesh=_mesh,
        compiler_params=_params,
        scratch_types=[
            pltpu.VMEM_SHARED((NP, DIM), jnp.float32),
            pltpu.VMEM((2, SBI, EB), jnp.int32),
            pltpu.VMEM((2, EB, DIM), jnp.float32),
            pltpu.VMEM((RCH,), jnp.float32),
            pltpu.SemaphoreType.DMA,
            pltpu.SemaphoreType.DMA,
            pltpu.SemaphoreType.DMA,
            pltpu.SemaphoreType.DMA,
        ],
    )
    def stage(tab, idx, dinv, out,
              accum, idx_buf, rows_buf, dinv_buf, gsem0, gsem1, ssem0, ssem1):
        c = lax.axis_index("c")
        s = lax.axis_index("s")
        row0 = s * ROWS_T
        gsems = (gsem0, gsem1)
        ssems = (ssem0, ssem1)
        sb0 = c * nsb_core + s * nsb

        def zrow(r, _):
            for w in range(DIM // 16):
                rows_buf[0, r, pl.ds(w * 16, 16)] = jnp.zeros(
                    (16,), jnp.float32)
            return 0
        lax.fori_loop(0, RCH, zrow, 0, unroll=8)

        def zcopy(kk, _):
            pltpu.sync_copy(rows_buf.at[0],
                            accum.at[pl.ds(row0 + kk * RCH, RCH)])
            return 0
        lax.fori_loop(0, NCH, zcopy, 0)
        plsc.subcore_barrier()

        def sb_body(sb, _):
            pltpu.sync_copy(idx.at[sb0 + sb], idx_buf)
            d = pltpu.async_copy(tab.at[idx_buf.at[0, 0]],
                                 rows_buf.at[0], gsem0)
            sdescs = [None] * SBI
            for j in range(SBI):
                slot = j % 2
                if j + 1 < SBI:
                    if j >= 1:
                        sdescs[j - 1].wait()
                    d_next = pltpu.async_copy(
                        tab.at[idx_buf.at[0, j + 1]],
                        rows_buf.at[(j + 1) % 2], gsems[(j + 1) % 2])
                d.wait()
                sd = pltpu.make_async_copy(
                    rows_buf.at[slot], accum.at[idx_buf.at[1, j]],
                    ssems[slot])
                sd.start(add=True)
                sdescs[j] = sd
                if j + 1 < SBI:
                    d = d_next
            sdescs[SBI - 2].wait()
            sdescs[SBI - 1].wait()
            return 0

        lax.fori_loop(0, nsb, sb_body, 0)
        plsc.subcore_barrier()

        def echunk(kk, _):
            pltpu.sync_copy(accum.at[pl.ds(row0 + kk * RCH, RCH)],
                            rows_buf.at[0])
            pltpu.sync_copy(
                dinv.at[pl.ds(c * NP + row0 + kk * RCH, RCH)], dinv_buf)

            def row_body(r, _):
                g = plsc.load_gather(
                    dinv_buf, [jnp.full((16,), r, jnp.int32)])
                for w in range(DIM // 16):
                    rows_buf[0, r, pl.ds(w * 16, 16)] = (
                        rows_buf[0, r, pl.ds(w * 16, 16)] * g)
                return 0

            lax.fori_loop(0, RCH, row_body, 0, unroll=8)
            pltpu.sync_copy(
                rows_buf.at[0],
                out.at[pl.ds(c * NP + row0 + kk * RCH, RCH)])
            return 0

        lax.fori_loop(0, NCH, echunk, 0)

    return stage


def _make_degree(n_edge_rows):
    et_rows = n_edge_rows // NS
    nsb = et_rows // SBI
    nsb_core = n_edge_rows // SBI

    @functools.partial(
        pl.kernel,
        out_type=jax.ShapeDtypeStruct((NC * NP, 16), jnp.float32),
        mesh=_mesh,
        compiler_params=_params,
        scratch_types=[
            pltpu.VMEM_SHARED((NP, 16), jnp.float32),
            pltpu.VMEM((2, SBI, EB), jnp.int32),
            pltpu.VMEM((EB, 16), jnp.float32),
            pltpu.SemaphoreType.DMA,
        ],
    )
    def degree(idx, out, dcum, idx_buf, ones_buf, dsem):
        c = lax.axis_index("c")
        s = lax.axis_index("s")
        row0 = s * ROWS_T
        sb0 = c * nsb_core + s * nsb

        def zrow(r, _):
            ones_buf[r, pl.ds(0, 16)] = jnp.zeros((16,), jnp.float32)
            return 0
        lax.fori_loop(0, EB, zrow, 0)

        def zcopy(kk, _):
            pltpu.sync_copy(ones_buf,
                            dcum.at[pl.ds(row0 + kk * EB, EB)])
            return 0
        lax.fori_loop(0, ROWS_T // EB, zcopy, 0)
        plsc.subcore_barrier()

        def orow(r, _):
            ones_buf[r, pl.ds(0, 16)] = jnp.ones((16,), jnp.float32)
            return 0
        lax.fori_loop(0, EB, orow, 0)

        def sb_body(sb, _):
            pltpu.sync_copy(idx.at[sb0 + sb], idx_buf)
            descs = []
            for j in range(SBI):
                sd = pltpu.make_async_copy(
                    ones_buf, dcum.at[idx_buf.at[1, j]], dsem)
                sd.start(add=True)
                descs.append(sd)
            for sd in descs:
                sd.wait()
            return 0

        lax.fori_loop(0, nsb, sb_body, 0)
        plsc.subcore_barrier()
        pltpu.sync_copy(dcum.at[pl.ds(row0, ROWS_T)],
                        out.at[pl.ds(c * NP + row0, ROWS_T)])

    return degree


def kernel(user_emb, item_emb, edge_user, edge_item):
    E = edge_user.shape[0]
    egran = NS * SBI * EB
    epad = ((E + egran - 1) // egran) * egran
    n_edge_rows = epad // EB

    eu = edge_user.astype(jnp.int32)
    ei = edge_item.astype(jnp.int32)
    pad = jnp.full((epad - E,), NU, jnp.int32)
    eu = jnp.concatenate([eu, pad])
    ei = jnp.concatenate([ei, pad])

    nsb_all = 2 * n_edge_rows // SBI

    def eidx(src_parts, dst_parts):
        src = jnp.concatenate(src_parts).reshape(nsb_all, 1, SBI, EB)
        dst = jnp.concatenate(dst_parts).reshape(nsb_all, 1, SBI, EB)
        return jnp.concatenate([src, dst], axis=1)

    idx_o = eidx([eu, ei + NP], [ei, eu])
    idx_e = eidx([ei, eu + NP], [eu, ei])

    degree_k = _make_degree(n_edge_rows)
    stage_k = _make_stage(n_edge_rows)

    deg = degree_k(idx_e)
    deg_u = jnp.where(deg[:NU, 0] == 0, 1.0, deg[:NU, 0])
    deg_i = jnp.where(deg[NP:NP + NU, 0] == 0, 1.0, deg[NP:NP + NU, 0])

    zpadn = jnp.zeros((NP - NU,), jnp.float32)
    dinv_u = jnp.concatenate([1.0 / deg_u, zpadn])
    dinv_i = jnp.concatenate([1.0 / deg_i, zpadn])
    dinv_o = jnp.concatenate([dinv_i, dinv_u])
    dinv_e = jnp.concatenate([dinv_u, dinv_i])

    isd_u = deg_u ** -0.5
    isd_i = deg_i ** -0.5
    zpadr = jnp.zeros((NP - NU, DIM), jnp.float32)
    x0 = jnp.concatenate([
        user_emb * isd_u[:, None], zpadr,
        item_emb * isd_i[:, None], zpadr,
    ], axis=0)

    y1 = stage_k(x0, idx_o, dinv_o)
    y2 = stage_k(y1, idx_e, dinv_e)
    y3 = stage_k(y2, idx_o, dinv_o)
    y4 = stage_k(y3, idx_e, dinv_e)

    p1 = y1[:NU]
    q1 = y1[NP:NP + NU]
    p2 = y2[:NU]
    q2 = y2[NP:NP + NU]
    p4 = y4[:NU]
    q4 = y4[NP:NP + NU]

    sd_u = jnp.sqrt(deg_u)[:, None]
    sd_i = jnp.sqrt(deg_i)[:, None]
    final_user = (2.0 * user_emb + sd_u * (q1 + 2.0 * p2 + p4)) / 3.0
    final_item = (2.0 * item_emb + sd_i * (p1 + 2.0 * q2 + q4)) / 3.0
    return (final_user, final_item)


# speedup vs baseline: 1.2001x; 1.1709x over previous
import functools

import jax
import jax.numpy as jnp
from jax import lax
from jax.experimental import pallas as pl
from jax.experimental.pallas import tpu as pltpu
from jax.experimental.pallas import tpu_sc as plsc

NU = 50000
DIM = 32
NC = 2
NS = 16
NP = 50176
ROWS_T = NP // NS
RCH = 64
NCH = ROWS_T // RCH
EB = 64
SBI = 16

_mesh = plsc.VectorSubcoreMesh(
    core_axis_name="c", subcore_axis_name="s", num_cores=NC, num_subcores=NS
)
_params = pltpu.CompilerParams(
    use_tc_tiling_on_sc=False, needs_layout_passes=False)


def _make_stage(n_edge_rows):
    et_rows = n_edge_rows // NS
    nsb = et_rows // SBI
    nsb_core = n_edge_rows // SBI

    @functools.partial(
        pl.kernel,
        out_type=jax.ShapeDtypeStruct((NC * NP, DIM), jnp.float32),
        mesh=_mesh,
        compiler_params=_params,
        scratch_types=[
            pltpu.VMEM_SHARED((NP, DIM), jnp.float32),
            pltpu.VMEM((2, SBI, EB), jnp.int32),
            pltpu.VMEM((3, EB, DIM), jnp.float32),
            pltpu.VMEM((RCH,), jnp.float32),
            pltpu.SemaphoreType.DMA,
            pltpu.SemaphoreType.DMA,
            pltpu.SemaphoreType.DMA,
            pltpu.SemaphoreType.DMA,
            pltpu.SemaphoreType.DMA,
            pltpu.SemaphoreType.DMA,
        ],
    )
    def stage(tab, idx, dinv, out,
              accum, idx_buf, rows_buf, dinv_buf,
              gsem0, gsem1, gsem2, ssem0, ssem1, ssem2):
        c = lax.axis_index("c")
        s = lax.axis_index("s")
        row0 = s * ROWS_T
        gsems = (gsem0, gsem1, gsem2)
        ssems = (ssem0, ssem1, ssem2)
        sb0 = c * nsb_core + s * nsb

        def zrow(r, _):
            for w in range(DIM // 16):
                rows_buf[0, r, pl.ds(w * 16, 16)] = jnp.zeros(
                    (16,), jnp.float32)
            return 0
        lax.fori_loop(0, RCH, zrow, 0, unroll=8)

        def zcopy(kk, _):
            pltpu.sync_copy(rows_buf.at[0],
                            accum.at[pl.ds(row0 + kk * RCH, RCH)])
            return 0
        lax.fori_loop(0, NCH, zcopy, 0)
        plsc.subcore_barrier()

        def sb_body(sb, _):
            pltpu.sync_copy(idx.at[sb0 + sb], idx_buf)
            gdescs = [None] * SBI
            sdescs = [None] * SBI
            for j in range(2):
                gdescs[j] = pltpu.async_copy(
                    tab.at[idx_buf.at[0, j]], rows_buf.at[j], gsems[j])
            for j in range(SBI):
                slot = j % 3
                if j + 2 < SBI:
                    if j >= 1:
                        sdescs[j - 1].wait()
                    gdescs[j + 2] = pltpu.async_copy(
                        tab.at[idx_buf.at[0, j + 2]],
                        rows_buf.at[(j + 2) % 3], gsems[(j + 2) % 3])
                gdescs[j].wait()
                sd = pltpu.make_async_copy(
                    rows_buf.at[slot], accum.at[idx_buf.at[1, j]],
                    ssems[slot])
                sd.start(add=True)
                sdescs[j] = sd
            sdescs[SBI - 3].wait()
            sdescs[SBI - 2].wait()
            sdescs[SBI - 1].wait()
            return 0

        lax.fori_loop(0, nsb, sb_body, 0)
        plsc.subcore_barrier()

        def echunk(kk, _):
            pltpu.sync_copy(accum.at[pl.ds(row0 + kk * RCH, RCH)],
                            rows_buf.at[0])
            pltpu.sync_copy(
                dinv.at[pl.ds(c * NP + row0 + kk * RCH, RCH)], dinv_buf)

            def row_body(r, _):
                g = plsc.load_gather(
                    dinv_buf, [jnp.full((16,), r, jnp.int32)])
                for w in range(DIM // 16):
                    rows_buf[0, r, pl.ds(w * 16, 16)] = (
                        rows_buf[0, r, pl.ds(w * 16, 16)] * g)
                return 0

            lax.fori_loop(0, RCH, row_body, 0, unroll=8)
            pltpu.sync_copy(
                rows_buf.at[0],
                out.at[pl.ds(c * NP + row0 + kk * RCH, RCH)])
            return 0

        lax.fori_loop(0, NCH, echunk, 0)

    return stage


def _make_degree(n_edge_rows):
    et_rows = n_edge_rows // NS
    nsb = et_rows // SBI
    nsb_core = n_edge_rows // SBI

    @functools.partial(
        pl.kernel,
        out_type=jax.ShapeDtypeStruct((NC * NP, 16), jnp.float32),
        mesh=_mesh,
        compiler_params=_params,
        scratch_types=[
            pltpu.VMEM_SHARED((NP, 16), jnp.float32),
            pltpu.VMEM((2, SBI, EB), jnp.int32),
            pltpu.VMEM((EB, 16), jnp.float32),
            pltpu.SemaphoreType.DMA,
        ],
    )
    def degree(idx, out, dcum, idx_buf, ones_buf, dsem):
        c = lax.axis_index("c")
        s = lax.axis_index("s")
        row0 = s * ROWS_T
        sb0 = c * nsb_core + s * nsb

        def zrow(r, _):
            ones_buf[r, pl.ds(0, 16)] = jnp.zeros((16,), jnp.float32)
            return 0
        lax.fori_loop(0, EB, zrow, 0)

        def zcopy(kk, _):
            pltpu.sync_copy(ones_buf,
                            dcum.at[pl.ds(row0 + kk * EB, EB)])
            return 0
        lax.fori_loop(0, ROWS_T // EB, zcopy, 0)
        plsc.subcore_barrier()

        def orow(r, _):
            ones_buf[r, pl.ds(0, 16)] = jnp.ones((16,), jnp.float32)
            return 0
        lax.fori_loop(0, EB, orow, 0)

        def sb_body(sb, _):
            pltpu.sync_copy(idx.at[sb0 + sb], idx_buf)
            descs = []
            for j in range(SBI):
                sd = pltpu.make_async_copy(
                    ones_buf, dcum.at[idx_buf.at[1, j]], dsem)
                sd.start(add=True)
                descs.append(sd)
            for sd in descs:
                sd.wait()
            return 0

        lax.fori_loop(0, nsb, sb_body, 0)
        plsc.subcore_barrier()
        pltpu.sync_copy(dcum.at[pl.ds(row0, ROWS_T)],
                        out.at[pl.ds(c * NP + row0, ROWS_T)])

    return degree


def kernel(user_emb, item_emb, edge_user, edge_item):
    E = edge_user.shape[0]
    egran = NS * SBI * EB
    epad = ((E + egran - 1) // egran) * egran
    n_edge_rows = epad // EB

    eu = edge_user.astype(jnp.int32)
    ei = edge_item.astype(jnp.int32)
    pad = jnp.full((epad - E,), NU, jnp.int32)
    eu = jnp.concatenate([eu, pad])
    ei = jnp.concatenate([ei, pad])

    nsb_all = 2 * n_edge_rows // SBI

    def eidx(src_parts, dst_parts):
        src = jnp.concatenate(src_parts).reshape(nsb_all, 1, SBI, EB)
        dst = jnp.concatenate(dst_parts).reshape(nsb_all, 1, SBI, EB)
        return jnp.concatenate([src, dst], axis=1)

    idx_o = eidx([eu, ei + NP], [ei, eu])
    idx_e = eidx([ei, eu + NP], [eu, ei])

    degree_k = _make_degree(n_edge_rows)
    stage_k = _make_stage(n_edge_rows)

    deg = degree_k(idx_e)
    deg_u = jnp.where(deg[:NU, 0] == 0, 1.0, deg[:NU, 0])
    deg_i = jnp.where(deg[NP:NP + NU, 0] == 0, 1.0, deg[NP:NP + NU, 0])

    zpadn = jnp.zeros((NP - NU,), jnp.float32)
    dinv_u = jnp.concatenate([1.0 / deg_u, zpadn])
    dinv_i = jnp.concatenate([1.0 / deg_i, zpadn])
    dinv_o = jnp.concatenate([dinv_i, dinv_u])
    dinv_e = jnp.concatenate([dinv_u, dinv_i])

    isd_u = deg_u ** -0.5
    isd_i = deg_i ** -0.5
    zpadr = jnp.zeros((NP - NU, DIM), jnp.float32)
    x0 = jnp.concatenate([
        user_emb * isd_u[:, None], zpadr,
        item_emb * isd_i[:, None], zpadr,
    ], axis=0)

    y1 = stage_k(x0, idx_o, dinv_o)
    y2 = stage_k(y1, idx_e, dinv_e)
    y3 = stage_k(y2, idx_o, dinv_o)
    y4 = stage_k(y3, idx_e, dinv_e)

    p1 = y1[:NU]
    q1 = y1[NP:NP + NU]
    p2 = y2[:NU]
    q2 = y2[NP:NP + NU]
    p4 = y4[:NU]
    q4 = y4[NP:NP + NU]

    sd_u = jnp.sqrt(deg_u)[:, None]
    sd_i = jnp.sqrt(deg_i)[:, None]
    final_user = (2.0 * user_emb + sd_u * (q1 + 2.0 * p2 + p4)) / 3.0
    final_item = (2.0 * item_emb + sd_i * (p1 + 2.0 * q2 + q4)) / 3.0
    return (final_user, final_item)
